# baseline (speedup 1.0000x reference)
import jax
import jax.numpy as jnp
from jax.experimental import pallas as pl
from jax.experimental.pallas import tpu as pltpu


_BS = 8


def _pad_body(in_ref, out_hbm, scratch, sems):
    i = pl.program_id(0)
    x = in_ref[...]
    xr = jnp.concatenate([x[:, -1:, :], x, x[:, :1, :]], axis=1)
    scratch[...] = jnp.concatenate([xr[:, :, -1:], xr, xr[:, :, :1]], axis=2)

    base = i * _BS
    copies = [
        pltpu.make_async_copy(
            scratch.at[b],
            out_hbm.at[base + b],
            sems.at[b],
        )
        for b in range(_BS)
    ]
    for c in copies:
        c.start()
    for c in copies:
        c.wait()


def kernel(inputs):
    b, h, w = inputs.shape
    return pl.pallas_call(
        _pad_body,
        grid=(b // _BS,),
        in_specs=[pl.BlockSpec((_BS, h, w), lambda i: (i, 0, 0))],
        out_specs=pl.BlockSpec(memory_space=pl.ANY),
        out_shape=jax.ShapeDtypeStruct((b, h + 2, w + 2), inputs.dtype),
        scratch_shapes=[
            pltpu.VMEM((_BS, h + 2, w + 2), inputs.dtype),
            pltpu.SemaphoreType.DMA((_BS,)),
        ],
    )(inputs)
